# Initial kernel scaffold; baseline (speedup 1.0000x reference)
import jax
import jax.numpy as jnp
from jax.experimental import pallas as pl


def kernel(emb, alpha, edge_index):
    raise NotImplementedError("write your pallas kernel here")


# speedup vs baseline: 7.4708x; 7.4708x over previous
import jax
import jax.numpy as jnp
from jax import lax
from jax.experimental import pallas as pl
from jax.experimental.pallas import tpu as pltpu
from jax.experimental.pallas import tpu_sc as plsc

_NUMS = 50000
_N = 2 * _NUMS
_D = 64
_E = 800000
_HALF = 32
_K = 128
_ECHUNKS = _E // _K
_NTEC = 16
_EK_PER_TEC = -(-_ECHUNKS // _NTEC)
_RCHUNKS = -(-_NUMS // _K)
_RK_PER_TEC = -(-_RCHUNKS // _NTEC)
_LAST_ROW_BASE = _NUMS - _K


def _row_base(c):
    return jnp.where(c == _RCHUNKS - 1, _LAST_ROW_BASE, c * _K)


def _sc_body(emb_cat, s_hbm, t_hbm, alpha_hbm, zrows_hbm, zvec_hbm, ones_hbm,
             xh_cat, out_cat,
             acc_sp, indeg_sp, outdeg_sp,
             sidx, tidx, gidx, rows, accb, ea, degb,
             zvec, onesv, av16):
    cid = lax.axis_index("c")
    wid = lax.axis_index("s")
    eoff = cid * _N
    xoff = cid * _NUMS
    eoffv = jnp.broadcast_to(eoff.astype(jnp.int32), (16,))
    xoffv = jnp.broadcast_to(xoff.astype(jnp.int32), (16,))

    pltpu.sync_copy(zvec_hbm, zvec)
    pltpu.sync_copy(ones_hbm, onesv)
    pltpu.sync_copy(alpha_hbm, av16)

    def zero_stripes(zero_counts):
        pltpu.sync_copy(zrows_hbm, rows)

        def body(k, carry):
            c = wid + _NTEC * k

            @pl.when(c < _RCHUNKS)
            def _():
                base = _row_base(c)
                pltpu.sync_copy(rows, acc_sp.at[pl.ds(base, _K)])
                if zero_counts:
                    pltpu.sync_copy(zvec, indeg_sp.at[pl.ds(base, _K)])
                    pltpu.sync_copy(zvec, outdeg_sp.at[pl.ds(base, _K)])
            return carry
        lax.fori_loop(0, _RK_PER_TEC, body, 0)

    def edge_pass(first):
        def body(k, carry):
            c = wid + _NTEC * k

            @pl.when(c < _ECHUNKS)
            def _():
                base = c * _K
                pltpu.sync_copy(s_hbm.at[pl.ds(base, _K)], sidx)
                pltpu.sync_copy(t_hbm.at[pl.ds(base, _K)], tidx)
                src, offv = (sidx, eoffv) if first else (tidx, xoffv)
                for g in range(_K // 16):
                    gs = pl.ds(g * 16, 16)
                    gidx[gs] = src[gs] + offv
                if first:
                    pltpu.sync_copy(emb_cat.at[gidx], rows)
                    pltpu.sync_copy(rows, acc_sp.at[tidx], add=True)
                    pltpu.sync_copy(onesv, indeg_sp.at[tidx], add=True)
                    pltpu.sync_copy(onesv, outdeg_sp.at[sidx], add=True)
                else:
                    pltpu.sync_copy(xh_cat.at[gidx], rows)
                    pltpu.sync_copy(rows, acc_sp.at[sidx], add=True)
            return carry
        lax.fori_loop(0, _EK_PER_TEC, body, 0)

    def normalize(deg_sp, final):
        def body(k, carry):
            c = wid + _NTEC * k

            @pl.when(c < _RCHUNKS)
            def _():
                base = _row_base(c)
                pltpu.sync_copy(acc_sp.at[pl.ds(base, _K)], accb)
                pltpu.sync_copy(deg_sp.at[pl.ds(base, _K)], degb)
                if final:
                    pltpu.sync_copy(xh_cat.at[pl.ds(xoff + base, _K)], ea)
                else:
                    pltpu.sync_copy(emb_cat.at[pl.ds(eoff + base, _K)], ea)
                    pltpu.sync_copy(
                        emb_cat.at[pl.ds(eoff + _NUMS + base, _K)], rows)
                if final:
                    av = av16[...]
                for g in range(_K // 16):
                    vg = 1.0 / (1.0 + degb[pl.ds(g * 16, 16)])
                    for j in range(16):
                        r = g * 16 + j
                        rcp = jnp.broadcast_to(vg[j], (16,))
                        for h in range(_HALF // 16):
                            cs = pl.ds(h * 16, 16)
                            if final:
                                x = ea[r, cs]
                                accb[r, cs] = av * x \
                                    + (x + accb[r, cs]) * rcp
                            else:
                                accb[r, cs] = (ea[r, cs] + accb[r, cs]) \
                                    * rcp + rows[r, cs]
                dst = out_cat if final else xh_cat
                pltpu.sync_copy(accb, dst.at[pl.ds(xoff + base, _K)])
            return carry
        lax.fori_loop(0, _RK_PER_TEC, body, 0)

    zero_stripes(zero_counts=True)
    plsc.subcore_barrier()
    edge_pass(first=True)
    plsc.subcore_barrier()
    normalize(indeg_sp, final=False)
    plsc.subcore_barrier()
    zero_stripes(zero_counts=False)
    plsc.subcore_barrier()
    edge_pass(first=False)
    plsc.subcore_barrier()
    normalize(outdeg_sp, final=True)


@jax.jit
def kernel(emb, alpha, edge_index):
    emb_cat = jnp.concatenate([emb[:, :_HALF], emb[:, _HALF:]], axis=0)
    s = edge_index[0]
    t = edge_index[1]
    alpha16 = jnp.broadcast_to(alpha, (16,)).astype(jnp.float32)
    zrows = jnp.zeros((_K, _HALF), jnp.float32)
    zvec = jnp.zeros((_K,), jnp.float32)
    ones = jnp.ones((_K,), jnp.float32)

    mesh = plsc.VectorSubcoreMesh(core_axis_name="c", subcore_axis_name="s")
    f = pl.kernel(
        _sc_body,
        out_type=[
            jax.ShapeDtypeStruct((2 * _NUMS, _HALF), jnp.float32),
            jax.ShapeDtypeStruct((2 * _NUMS, _HALF), jnp.float32),
        ],
        mesh=mesh,
        compiler_params=pltpu.CompilerParams(
            needs_layout_passes=False, use_tc_tiling_on_sc=False),
        scratch_types=[
            pltpu.VMEM_SHARED((_NUMS, _HALF), jnp.float32),
            pltpu.VMEM_SHARED((_NUMS,), jnp.float32),
            pltpu.VMEM_SHARED((_NUMS,), jnp.float32),
            pltpu.VMEM((_K,), jnp.int32),
            pltpu.VMEM((_K,), jnp.int32),
            pltpu.VMEM((_K,), jnp.int32),
            pltpu.VMEM((_K, _HALF), jnp.float32),
            pltpu.VMEM((_K, _HALF), jnp.float32),
            pltpu.VMEM((_K, _HALF), jnp.float32),
            pltpu.VMEM((_K,), jnp.float32),
            pltpu.VMEM((_K,), jnp.float32),
            pltpu.VMEM((_K,), jnp.float32),
            pltpu.VMEM((16,), jnp.float32),
        ],
    )
    xh_cat, out_cat = f(emb_cat, s, t, alpha16, zrows, zvec, ones)
    lo = jnp.concatenate([out_cat[:_NUMS], out_cat[_NUMS:]], axis=1)
    return jnp.concatenate([lo, jnp.zeros((_NUMS, _D), jnp.float32)], axis=0)
